# baseline (speedup 1.0000x reference)
import jax
import jax.numpy as jnp
from jax.experimental import pallas as pl
from jax.experimental.pallas import tpu as pltpu

_BB = 1024


def _vad_block(audio_ref, h0_ref, c0_ref, wstft_ref, w0stft_ref,
               bigw1_ref, b1_ref, w2_ref, b2_ref, w3_ref, b3_ref,
               w4_ref, b4_ref, wl_ref, bl_ref, ow_ref, ob_ref,
               fin_ref, hout_ref, cout_ref):
    h = h0_ref[...].T
    c = c0_ref[...].T
    bb = h.shape[1]
    acc = jnp.ones((1, bb), jnp.float32)
    wt = wstft_ref[...]
    w0 = w0stft_ref[...]
    bigw1 = bigw1_ref[...]
    w2 = w2_ref[...]
    w3 = w3_ref[...]
    w4 = w4_ref[...]
    wl = wl_ref[...]

    bf16 = jnp.bfloat16

    def mcat_of(i):
        base = 512 * i
        y = audio_ref[base:base + 576, :].astype(bf16)
        mags = []
        for (woff, wmat) in ((0, w0), (64, wt), (192, wt), (320, wt)):
            win = y[woff:woff + 256]
            s = jnp.dot(wmat, win, preferred_element_type=jnp.float32)
            m2 = s[0:136] ** 2 + s[136:272] ** 2
            mags.append(m2 * jax.lax.rsqrt(jnp.maximum(m2, 1e-37)))
        return jnp.concatenate(mags, axis=0).astype(bf16)

    def layer(wref, bref, xs):
        return [jnp.maximum(
            jnp.dot(wref, x, preferred_element_type=jnp.float32)
            + bref[...], 0.0).astype(bf16) for x in xs]

    def lstm_step(x_t, h, c):
        xin = jnp.concatenate([x_t, h.astype(bf16)], axis=0)
        gates = jnp.dot(wl, xin, preferred_element_type=jnp.float32) \
            + bl_ref[...]
        i_g = jax.nn.sigmoid(gates[0:128])
        f_g = jax.nn.sigmoid(gates[128:256])
        g_g = jnp.tanh(gates[256:384])
        o_g = jax.nn.sigmoid(gates[384:512])
        c = f_g * c + i_g * g_g
        h = o_g * jnp.tanh(c)
        return h, c

    mc0 = [mcat_of(i) for i in range(4)]
    g0 = layer(w4, b4_ref,
               layer(w3, b3_ref,
                     layer(w2, b2_ref,
                           layer(bigw1, b1_ref, mc0))))
    hs = []
    h, c = lstm_step(g0[0], h, c)
    hs.append(h)
    mc1a = [mcat_of(i) for i in (4, 5)]
    h, c = lstm_step(g0[1], h, c)
    hs.append(h)
    mc1b = [mcat_of(i) for i in (6, 7)]
    h, c = lstm_step(g0[2], h, c)
    hs.append(h)
    g1 = layer(bigw1, b1_ref, mc1a + mc1b)
    h, c = lstm_step(g0[3], h, c)
    hs.append(h)
    g1 = layer(w2, b2_ref, g1)
    g1 = layer(w3, b3_ref, g1)
    g1 = layer(w4, b4_ref, g1)
    for i in range(4):
        h, c = lstm_step(g1[i], h, c)
        hs.append(h)

    hcat = jnp.concatenate([hh.astype(bf16) for hh in hs], axis=0)
    pall = jnp.dot(ow_ref[...], hcat, preferred_element_type=jnp.float32) \
        + ob_ref[...]
    q = 1.0 - jax.nn.sigmoid(pall)
    q = q[0:4] * q[4:8]
    q = q[0:2] * q[2:4]
    fin_ref[...] = 1.0 - q[0:1] * q[1:2]
    hout_ref[...] = h.T
    cout_ref[...] = c.T


def kernel(audio_input, hidden_state, cell_state, stft_w,
           enc_w1, enc_b1, enc_w2, enc_b2, enc_w3, enc_b3, enc_w4, enc_b4,
           w_ih, w_hh, b_ih, b_hh, out_w, out_b):
    f32 = jnp.float32
    b = audio_input.shape[0]

    audio_t = audio_input.T

    wt2 = stft_w[:, 0, :].T
    c0m = wt2[64:256].at[1:65].add(jnp.flip(wt2[0:64], axis=0))

    def bank_rows(wcols, klen):
        out = jnp.zeros((272, 256), f32)
        out = out.at[0:129, 0:klen].set(wcols[:, 0:129].T)
        out = out.at[136:265, 0:klen].set(wcols[:, 129:258].T)
        return out

    bf16 = jnp.bfloat16
    wstft = bank_rows(wt2, 256).astype(bf16)
    w0stft = bank_rows(c0m, 192).astype(bf16)

    bigw1 = jnp.zeros((512, 544), f32)
    for t in range(4):
        for s in range(4):
            d = s - t + 1
            if 0 <= d <= 2:
                bigw1 = bigw1.at[128 * t:128 * (t + 1),
                                 136 * s:136 * s + 129].set(enc_w1[:, :, d])
    bigw1 = bigw1.astype(bf16)
    b1b = jnp.broadcast_to(jnp.tile(enc_b1, 4)[:, None], (512, _BB))

    z64 = jnp.zeros((64, 128), f32)
    w2t = jnp.concatenate([
        jnp.concatenate([enc_w2[:, :, 1], enc_w2[:, :, 2], z64, z64], axis=1),
        jnp.concatenate([z64, enc_w2[:, :, 0], enc_w2[:, :, 1],
                         enc_w2[:, :, 2]], axis=1),
    ], axis=0).astype(bf16)
    b2b = jnp.broadcast_to(
        jnp.concatenate([enc_b2, enc_b2])[:, None], (128, _BB))

    w3t = jnp.concatenate([enc_w3[:, :, 1], enc_w3[:, :, 2]],
                          axis=1).astype(bf16)
    b3b = jnp.broadcast_to(enc_b3[:, None], (64, _BB))

    w4t = enc_w4[:, :, 1].astype(bf16)
    b4b = jnp.broadcast_to(enc_b4[:, None], (128, _BB))

    wlt = jnp.concatenate([w_ih, w_hh], axis=1).astype(bf16)
    blb = jnp.broadcast_to((b_ih + b_hh)[:, None], (512, _BB))
    ow8 = jnp.zeros((8, 1024), f32)
    for i in range(8):
        ow8 = ow8.at[i, 128 * i:128 * (i + 1)].set(out_w[0])
    ow8 = ow8.astype(bf16)
    ob8 = jnp.broadcast_to(out_b[:, None], (8, _BB))

    grid = (b // _BB,)

    def bcast(shape):
        nd = len(shape)
        return pl.BlockSpec(shape, lambda i: (0,) * nd)

    fin_t, h_fin_t, c_fin_t = pl.pallas_call(
        _vad_block,
        grid=grid,
        in_specs=[
            pl.BlockSpec((4160, _BB), lambda i: (0, i)),
            pl.BlockSpec((_BB, 128), lambda i: (i, 0)),
            pl.BlockSpec((_BB, 128), lambda i: (i, 0)),
            bcast((272, 256)),
            bcast((272, 256)),
            bcast((512, 544)),
            bcast((512, _BB)),
            bcast((128, 512)),
            bcast((128, _BB)),
            bcast((64, 128)),
            bcast((64, _BB)),
            bcast((128, 64)),
            bcast((128, _BB)),
            bcast((512, 256)),
            bcast((512, _BB)),
            bcast((8, 1024)),
            bcast((8, _BB)),
        ],
        out_specs=[
            pl.BlockSpec((1, _BB), lambda i: (0, i)),
            pl.BlockSpec((_BB, 128), lambda i: (i, 0)),
            pl.BlockSpec((_BB, 128), lambda i: (i, 0)),
        ],
        out_shape=[
            jax.ShapeDtypeStruct((1, b), f32),
            jax.ShapeDtypeStruct((b, 128), f32),
            jax.ShapeDtypeStruct((b, 128), f32),
        ],
        compiler_params=pltpu.CompilerParams(
            dimension_semantics=("parallel",),
            vmem_limit_bytes=60 * 1024 * 1024,
        ),
        name="vad256ms_fused",
    )(audio_t, hidden_state, cell_state, wstft, w0stft,
      bigw1, b1b, w2t, b2b, w3t, b3b, w4t, b4b, wlt, blb, ow8, ob8)

    return fin_t.reshape(b, 1, 1), h_fin_t, c_fin_t


# speedup vs baseline: 1.0365x; 1.0365x over previous
import jax
import jax.numpy as jnp
from jax.experimental import pallas as pl
from jax.experimental.pallas import tpu as pltpu

_BB = 1024


def _vad_block(audio_ref, h0_ref, c0_ref, wstft_ref, w0stft_ref,
               bigw1_ref, b1_ref, w2_ref, b2_ref, w3_ref, b3_ref,
               w4_ref, b4_ref, wl_ref, bl_ref, ow_ref, ob_ref,
               fin_ref, hout_ref, cout_ref):
    h = h0_ref[...].T
    c = c0_ref[...].T
    bb = h.shape[1]
    acc = jnp.ones((1, bb), jnp.float32)
    wt = wstft_ref[...]
    w0 = w0stft_ref[...]
    bigw1 = bigw1_ref[...]
    w2 = w2_ref[...]
    w3 = w3_ref[...]
    w4 = w4_ref[...]
    wl = wl_ref[...]

    bf16 = jnp.bfloat16

    def mcat_of(i):
        base = 512 * i
        y = audio_ref[base:base + 576, :].astype(bf16)
        mags = []
        for (woff, wmat) in ((0, w0), (64, wt), (192, wt), (320, wt)):
            win = y[woff:woff + 256]
            s = jnp.dot(wmat, win, preferred_element_type=jnp.float32)
            m2 = s[0:136] ** 2 + s[136:272] ** 2
            mags.append(m2 * jax.lax.rsqrt(jnp.maximum(m2, 1e-37)))
        return jnp.concatenate(mags, axis=0).astype(bf16)

    def layer(wref, bref, xs):
        return [jnp.maximum(
            jnp.dot(wref, x, preferred_element_type=jnp.float32)
            + bref[...], 0.0).astype(bf16) for x in xs]

    def lstm_step(x_t, h, c, acc):
        xin = jnp.concatenate([x_t, h.astype(bf16)], axis=0)
        gates = jnp.dot(wl, xin, preferred_element_type=jnp.float32) \
            + bl_ref[...]
        i_g = jax.nn.sigmoid(gates[0:128])
        f_g = jax.nn.sigmoid(gates[128:256])
        g_g = jnp.tanh(gates[256:384])
        o_g = jax.nn.sigmoid(gates[384:512])
        c = f_g * c + i_g * g_g
        h = o_g * jnp.tanh(c)
        p = jax.nn.sigmoid(
            jnp.sum(h * ow_ref[...], axis=0, keepdims=True) + ob_ref[...])
        return h, c, acc * (1.0 - p)

    mc0 = [mcat_of(i) for i in range(4)]
    g0 = layer(w4, b4_ref,
               layer(w3, b3_ref,
                     layer(w2, b2_ref,
                           layer(bigw1, b1_ref, mc0))))
    h, c, acc = lstm_step(g0[0], h, c, acc)
    mc1a = [mcat_of(i) for i in (4, 5)]
    h, c, acc = lstm_step(g0[1], h, c, acc)
    mc1b = [mcat_of(i) for i in (6, 7)]
    h, c, acc = lstm_step(g0[2], h, c, acc)
    g1 = layer(bigw1, b1_ref, mc1a + mc1b)
    h, c, acc = lstm_step(g0[3], h, c, acc)
    g1 = layer(w2, b2_ref, g1)
    g1 = layer(w3, b3_ref, g1)
    g1 = layer(w4, b4_ref, g1)
    for i in range(4):
        h, c, acc = lstm_step(g1[i], h, c, acc)

    fin_ref[...] = 1.0 - acc
    hout_ref[...] = h.T
    cout_ref[...] = c.T


def kernel(audio_input, hidden_state, cell_state, stft_w,
           enc_w1, enc_b1, enc_w2, enc_b2, enc_w3, enc_b3, enc_w4, enc_b4,
           w_ih, w_hh, b_ih, b_hh, out_w, out_b):
    f32 = jnp.float32
    b = audio_input.shape[0]

    audio_t = audio_input.T

    wt2 = stft_w[:, 0, :].T
    c0m = wt2[64:256].at[1:65].add(jnp.flip(wt2[0:64], axis=0))

    def bank_rows(wcols, klen):
        out = jnp.zeros((272, 256), f32)
        out = out.at[0:129, 0:klen].set(wcols[:, 0:129].T)
        out = out.at[136:265, 0:klen].set(wcols[:, 129:258].T)
        return out

    bf16 = jnp.bfloat16
    wstft = bank_rows(wt2, 256).astype(bf16)
    w0stft = bank_rows(c0m, 192).astype(bf16)

    bigw1 = jnp.zeros((512, 544), f32)
    for t in range(4):
        for s in range(4):
            d = s - t + 1
            if 0 <= d <= 2:
                bigw1 = bigw1.at[128 * t:128 * (t + 1),
                                 136 * s:136 * s + 129].set(enc_w1[:, :, d])
    bigw1 = bigw1.astype(bf16)
    b1b = jnp.broadcast_to(jnp.tile(enc_b1, 4)[:, None], (512, _BB))

    z64 = jnp.zeros((64, 128), f32)
    w2t = jnp.concatenate([
        jnp.concatenate([enc_w2[:, :, 1], enc_w2[:, :, 2], z64, z64], axis=1),
        jnp.concatenate([z64, enc_w2[:, :, 0], enc_w2[:, :, 1],
                         enc_w2[:, :, 2]], axis=1),
    ], axis=0).astype(bf16)
    b2b = jnp.broadcast_to(
        jnp.concatenate([enc_b2, enc_b2])[:, None], (128, _BB))

    w3t = jnp.concatenate([enc_w3[:, :, 1], enc_w3[:, :, 2]],
                          axis=1).astype(bf16)
    b3b = jnp.broadcast_to(enc_b3[:, None], (64, _BB))

    w4t = enc_w4[:, :, 1].astype(bf16)
    b4b = jnp.broadcast_to(enc_b4[:, None], (128, _BB))

    wlt = jnp.concatenate([w_ih, w_hh], axis=1).astype(bf16)
    blb = jnp.broadcast_to((b_ih + b_hh)[:, None], (512, _BB))
    owb = jnp.broadcast_to(out_w.T, (128, _BB))
    obb = jnp.broadcast_to(out_b[:, None], (1, _BB))

    grid = (b // _BB,)

    def bcast(shape):
        nd = len(shape)
        return pl.BlockSpec(shape, lambda i: (0,) * nd)

    fin_t, h_fin_t, c_fin_t = pl.pallas_call(
        _vad_block,
        grid=grid,
        in_specs=[
            pl.BlockSpec((4160, _BB), lambda i: (0, i)),
            pl.BlockSpec((_BB, 128), lambda i: (i, 0)),
            pl.BlockSpec((_BB, 128), lambda i: (i, 0)),
            bcast((272, 256)),
            bcast((272, 256)),
            bcast((512, 544)),
            bcast((512, _BB)),
            bcast((128, 512)),
            bcast((128, _BB)),
            bcast((64, 128)),
            bcast((64, _BB)),
            bcast((128, 64)),
            bcast((128, _BB)),
            bcast((512, 256)),
            bcast((512, _BB)),
            bcast((128, _BB)),
            bcast((1, _BB)),
        ],
        out_specs=[
            pl.BlockSpec((1, _BB), lambda i: (0, i)),
            pl.BlockSpec((_BB, 128), lambda i: (i, 0)),
            pl.BlockSpec((_BB, 128), lambda i: (i, 0)),
        ],
        out_shape=[
            jax.ShapeDtypeStruct((1, b), f32),
            jax.ShapeDtypeStruct((b, 128), f32),
            jax.ShapeDtypeStruct((b, 128), f32),
        ],
        compiler_params=pltpu.CompilerParams(
            dimension_semantics=("parallel",),
            vmem_limit_bytes=60 * 1024 * 1024,
        ),
        name="vad256ms_fused",
    )(audio_t, hidden_state, cell_state, wstft, w0stft,
      bigw1, b1b, w2t, b2b, w3t, b3b, w4t, b4b, wlt, blb, owb, obb)

    return fin_t.reshape(b, 1, 1), h_fin_t, c_fin_t
